# baseline (speedup 1.0000x reference)
import functools

import jax
import jax.numpy as jnp
from jax.experimental import pallas as pl
from jax.experimental.pallas import tpu as pltpu

EPS = 1e-06
K = 32
INF = 3.0e38
IBIG = 2**31 - 1


def _stats_body(ref_ref, sum_ref, sq_ref):
    i = pl.program_id(0)
    blk = ref_ref[...]
    s = jnp.sum(blk, axis=0, keepdims=True)
    q = jnp.sum(blk * blk, axis=0, keepdims=True)

    @pl.when(i == 0)
    def _():
        sum_ref[...] = s
        sq_ref[...] = q

    @pl.when(i > 0)
    def _():
        sum_ref[...] += s
        sq_ref[...] += q


def _main_body(tgt_ref, ref_ref, mean_ref, std_ref, dom_ref,
               topd_ref, topi_ref, topdom_ref, strip_ref,
               *, n_chunks, chunk, br):
    n = n_chunks * chunk
    mean = mean_ref[...]
    std = std_ref[...]
    st = (tgt_ref[...] - mean) / std
    t2 = jnp.sum(st * st, axis=1, keepdims=True)

    for c in range(n_chunks):
        src = (ref_ref[pl.ds(c * chunk, chunk), :] - mean) / std
        r2 = jnp.sum(src * src, axis=1)[None, :]
        dots = jax.lax.dot_general(
            st, src, (((1,), (1,)), ((), ())),
            preferred_element_type=jnp.float32)
        d2 = t2 + r2 - 2.0 * dots
        strip_ref[:, pl.ds(c * chunk, chunk)] = jnp.sqrt(
            jnp.maximum(d2, 1e-12))

    lane_k = jax.lax.broadcasted_iota(jnp.int32, (br, K), 1)

    def round_body(k, carry):
        dacc, iacc, domacc = carry

        def find_chunk(c, fc):
            m, idxm = fc
            s = strip_ref[:, pl.ds(c * chunk, chunk)]
            cols = jax.lax.broadcasted_iota(jnp.int32, (br, chunk), 1) \
                + c * chunk
            m_c = jnp.min(s, axis=1)
            i_c = jnp.min(jnp.where(s == m_c[:, None], cols, IBIG), axis=1)
            better = m_c < m
            m = jnp.where(better, m_c, m)
            idxm = jnp.where(better, i_c, idxm)
            return m, idxm

        m0 = jnp.full((br,), INF, jnp.float32)
        i0 = jnp.full((br,), IBIG, jnp.int32)
        m, idxm = jax.lax.fori_loop(0, n_chunks, find_chunk, (m0, i0))

        def commit_chunk(c, dval):
            s = strip_ref[:, pl.ds(c * chunk, chunk)]
            cols = jax.lax.broadcasted_iota(jnp.int32, (br, chunk), 1) \
                + c * chunk
            eq = cols == idxm[:, None]
            strip_ref[:, pl.ds(c * chunk, chunk)] = jnp.where(eq, INF, s)
            domc = dom_ref[:, pl.ds(c * chunk, chunk)]
            dval = dval + jnp.sum(jnp.where(eq, domc, 0.0), axis=1)
            return dval

        dval = jax.lax.fori_loop(0, n_chunks, commit_chunk,
                                 jnp.zeros((br,), jnp.float32))

        sel = lane_k == k
        dacc = jnp.where(sel, m[:, None], dacc)
        iacc = jnp.where(sel, idxm[:, None], iacc)
        domacc = jnp.where(sel, dval[:, None], domacc)
        return dacc, iacc, domacc

    dacc = jnp.zeros((br, K), jnp.float32)
    iacc = jnp.zeros((br, K), jnp.int32)
    domacc = jnp.zeros((br, K), jnp.float32)
    dacc, iacc, domacc = jax.lax.fori_loop(
        0, K, round_body, (dacc, iacc, domacc))
    topd_ref[...] = dacc
    topi_ref[...] = iacc
    topdom_ref[...] = domacc


def _gather_body(topi_ref, tgt_ref, ref_ref, topd_ref, topdom_ref, tdom_ref,
                 proto_ref, delta_ref, absd_ref, meand_ref, stdd_ref,
                 w_ref, same_ref, pscr_ref, wscr_ref, *, br):
    d = topd_ref[...]
    rw = 1.0 / jnp.maximum(d, EPS)
    w = rw / jnp.sum(rw, axis=1, keepdims=True)
    w_ref[...] = w
    wscr_ref[...] = w

    def row_body(r, _):
        acc = jnp.zeros((1, tgt_ref.shape[1]), jnp.float32)
        for k in range(K):
            idx = topi_ref[r, k]
            wk = wscr_ref[r, k]
            acc = acc + wk * ref_ref[pl.ds(idx, 1), :]
        pscr_ref[pl.ds(r, 1), :] = acc
        return 0

    jax.lax.fori_loop(0, br, row_body, 0)

    proto = pscr_ref[...]
    tgt = tgt_ref[...]
    delta = tgt - proto
    proto_ref[...] = proto
    delta_ref[...] = delta
    absd_ref[...] = jnp.abs(delta)
    meand = jnp.sum(w * d, axis=1, keepdims=True)
    meand_ref[...] = meand
    stdd_ref[...] = jnp.sqrt(jnp.maximum(
        jnp.sum(w * jnp.square(d - meand), axis=1, keepdims=True), 1e-12))
    same = (topdom_ref[...] == tdom_ref[...]).astype(jnp.float32)
    same_ref[...] = jnp.mean(same, axis=1, keepdims=True)


def kernel(target_embeddings, reference_embeddings, target_domains,
           reference_domains, exclude_self):
    Q, D = target_embeddings.shape
    N, _ = reference_embeddings.shape
    NB = 16
    ssum, ssq = pl.pallas_call(
        _stats_body,
        grid=(NB,),
        in_specs=[pl.BlockSpec((N // NB, D), lambda i: (i, 0))],
        out_specs=(
            pl.BlockSpec((1, D), lambda i: (0, 0)),
            pl.BlockSpec((1, D), lambda i: (0, 0)),
        ),
        out_shape=(
            jax.ShapeDtypeStruct((1, D), jnp.float32),
            jax.ShapeDtypeStruct((1, D), jnp.float32),
        ),
    )(reference_embeddings)
    mean = ssum / N
    var = jnp.maximum(ssq / N - mean * mean, 0.0)
    std = jnp.sqrt(var)
    std = jnp.where(std > 0, std, jnp.ones_like(std))

    BR = min(64, Q)
    GB = Q // BR
    CH = max(512, N // 32)
    NCH = N // CH
    dom_f32 = reference_domains.astype(jnp.float32).reshape(1, N)
    tdom_f32 = target_domains.astype(jnp.float32).reshape(Q, 1)

    topd, topi, topdom = pl.pallas_call(
        functools.partial(_main_body, n_chunks=NCH, chunk=CH, br=BR),
        grid=(GB,),
        in_specs=[
            pl.BlockSpec((BR, D), lambda i: (i, 0)),
            pl.BlockSpec((N, D), lambda i: (0, 0)),
            pl.BlockSpec((1, D), lambda i: (0, 0)),
            pl.BlockSpec((1, D), lambda i: (0, 0)),
            pl.BlockSpec((1, N), lambda i: (0, 0)),
        ],
        out_specs=(
            pl.BlockSpec((BR, K), lambda i: (i, 0)),
            pl.BlockSpec((BR, K), lambda i: (i, 0)),
            pl.BlockSpec((BR, K), lambda i: (i, 0)),
        ),
        out_shape=(
            jax.ShapeDtypeStruct((Q, K), jnp.float32),
            jax.ShapeDtypeStruct((Q, K), jnp.int32),
            jax.ShapeDtypeStruct((Q, K), jnp.float32),
        ),
        scratch_shapes=[pltpu.VMEM((BR, N), jnp.float32)],
    )(target_embeddings, reference_embeddings, mean, std, dom_f32)

    outs = pl.pallas_call(
        functools.partial(_gather_body, br=BR),
        grid=(GB,),
        in_specs=[
            pl.BlockSpec((BR, K), lambda i: (i, 0)),
            pl.BlockSpec((BR, D), lambda i: (i, 0)),
            pl.BlockSpec((N, D), lambda i: (0, 0)),
            pl.BlockSpec((BR, K), lambda i: (i, 0)),
            pl.BlockSpec((BR, K), lambda i: (i, 0)),
            pl.BlockSpec((BR, 1), lambda i: (i, 0)),
        ],
        out_specs=(
            pl.BlockSpec((BR, D), lambda i: (i, 0)),
            pl.BlockSpec((BR, D), lambda i: (i, 0)),
            pl.BlockSpec((BR, D), lambda i: (i, 0)),
            pl.BlockSpec((BR, 1), lambda i: (i, 0)),
            pl.BlockSpec((BR, 1), lambda i: (i, 0)),
            pl.BlockSpec((BR, K), lambda i: (i, 0)),
            pl.BlockSpec((BR, 1), lambda i: (i, 0)),
        ),
        out_shape=(
            jax.ShapeDtypeStruct((Q, D), jnp.float32),
            jax.ShapeDtypeStruct((Q, D), jnp.float32),
            jax.ShapeDtypeStruct((Q, D), jnp.float32),
            jax.ShapeDtypeStruct((Q, 1), jnp.float32),
            jax.ShapeDtypeStruct((Q, 1), jnp.float32),
            jax.ShapeDtypeStruct((Q, K), jnp.float32),
            jax.ShapeDtypeStruct((Q, 1), jnp.float32),
        ),
        scratch_shapes=[pltpu.VMEM((BR, D), jnp.float32),
                        pltpu.VMEM((BR, K), jnp.float32)],
    )(topi, target_embeddings, reference_embeddings, topd, topdom, tdom_f32)
    proto, delta, absd, meand, stdd, w, same = outs

    return (proto, delta, absd, topd[:, 0], meand[:, 0], stdd[:, 0],
            topi, topd, w, same[:, 0])
